# baseline (speedup 1.0000x reference)
import functools

import jax
import jax.numpy as jnp
from jax import lax
from jax.experimental import pallas as pl
from jax.experimental.pallas import tpu as pltpu
from jax.experimental.pallas import tpu_sc as plsc

NUM_CORES = 2
NUM_SUBCORES = 16
LANES = 16
NW = NUM_CORES * NUM_SUBCORES

BATCH = 16384
EMBED_K = 64
BPW = BATCH // NW


def _sc_body(users_h, items_h, gu_h, gi_h, xui_h, gamma_u_h, gamma_i_h,
             idx_u, idx_i, u_rows, i_rows, xui_v,
             sem_u, sem_i, sem_ou, sem_oi):
    wid = lax.axis_index("s") * NUM_CORES + lax.axis_index("c")
    base = wid * BPW

    pltpu.sync_copy(users_h.at[pl.ds(base, BPW)], idx_u)
    pltpu.sync_copy(items_h.at[pl.ds(base, BPW)], idx_i)

    cu = pltpu.async_copy(gu_h.at[idx_u], u_rows, sem_u)
    ci = pltpu.async_copy(gi_h.at[idx_i], i_rows, sem_i)
    cu.wait()
    ci.wait()

    ou = pltpu.async_copy(u_rows, gamma_u_h.at[pl.ds(base, BPW)], sem_ou)
    oi = pltpu.async_copy(i_rows, gamma_i_h.at[pl.ds(base, BPW)], sem_oi)

    lane = lax.iota(jnp.int32, LANES)

    def g_body(g, _):
        acc = jnp.zeros((LANES,), jnp.float32)
        row_idx = lane + g * LANES
        col = jnp.zeros((LANES,), jnp.int32)
        for _k in range(EMBED_K):
            uu = plsc.load_gather(u_rows, [row_idx, col])
            ii = plsc.load_gather(i_rows, [row_idx, col])
            acc = acc + uu * ii
            col = col + 1
        xui_v[pl.ds(g * LANES, LANES)] = acc
        return _

    lax.fori_loop(0, BPW // LANES, g_body, 0, unroll=False)

    pltpu.sync_copy(xui_v, xui_h.at[pl.ds(base, BPW)])
    ou.wait()
    oi.wait()


@jax.jit
def _rslogic2_sc(users, items, Gu, Gi):
    mesh = plsc.VectorSubcoreMesh(
        core_axis_name="c", subcore_axis_name="s",
        num_cores=NUM_CORES, num_subcores=NUM_SUBCORES)
    return pl.kernel(
        _sc_body,
        out_type=(
            jax.ShapeDtypeStruct((BATCH,), jnp.float32),
            jax.ShapeDtypeStruct((BATCH, EMBED_K), jnp.float32),
            jax.ShapeDtypeStruct((BATCH, EMBED_K), jnp.float32),
        ),
        mesh=mesh,
        compiler_params=pltpu.CompilerParams(needs_layout_passes=False, use_tc_tiling_on_sc=False),
        scratch_types=[
            pltpu.VMEM((BPW,), jnp.int32),
            pltpu.VMEM((BPW,), jnp.int32),
            pltpu.VMEM((BPW, EMBED_K), jnp.float32),
            pltpu.VMEM((BPW, EMBED_K), jnp.float32),
            pltpu.VMEM((BPW,), jnp.float32),
            pltpu.SemaphoreType.DMA,
            pltpu.SemaphoreType.DMA,
            pltpu.SemaphoreType.DMA,
            pltpu.SemaphoreType.DMA,
        ],
    )(users, items, Gu, Gi)


def kernel(users, items, Gu, Gi):
    xui, gamma_u, gamma_i = _rslogic2_sc(users, items, Gu, Gi)
    return (xui, gamma_u, gamma_i)
